# baseline (speedup 1.0000x reference)
import functools

import jax
import jax.numpy as jnp
from jax import lax
from jax.experimental import pallas as pl
from jax.experimental.pallas import tpu as pltpu
from jax.experimental.pallas import tpu_sc as plsc

N_NODES = 10000
N_EDGES = 320000
D = 128

NC = 2
NS = 16
EDGES_PER_CORE = N_EDGES // NC
EDGES_PER_TILE = EDGES_PER_CORE // NS
CHUNK = 80
N_CHUNKS = EDGES_PER_TILE // CHUNK
ROWS_PER_TILE = 632
N_PAD = ROWS_PER_TILE * NS


def _make_sc_agg(compute_deg: bool):
    mesh = plsc.VectorSubcoreMesh(core_axis_name="c", subcore_axis_name="s")

    out_type = [jax.ShapeDtypeStruct((NC * N_PAD, D), jnp.float32)]
    if compute_deg:
        out_type.append(jax.ShapeDtypeStruct((NC, N_PAD), jnp.float32))

    scratch = [
        pltpu.VMEM_SHARED((N_PAD, D), jnp.float32),
        pltpu.VMEM((EDGES_PER_TILE,), jnp.int32),
        pltpu.VMEM((N_CHUNKS, CHUNK), jnp.int32),
        pltpu.VMEM((CHUNK, D), jnp.float32),
        pltpu.VMEM((CHUNK, D), jnp.float32),
        pltpu.SemaphoreType.DMA,
        pltpu.SemaphoreType.DMA,
    ]
    if compute_deg:
        scratch += [
            pltpu.VMEM_SHARED((N_PAD,), jnp.float32),
            pltpu.VMEM((CHUNK,), jnp.float32),
        ]

    def body(x_hbm, src_hbm, dst_hbm, zeros_hbm, zdeg_hbm, part_hbm, *rest):
        if compute_deg:
            deg_hbm = rest[0]
            acc_sh, srct_v, dstt_v, rows_a, rows_b, sem, sem_s, deg_sh, ones_v = rest[1:]
        else:
            (acc_sh, srct_v, dstt_v, rows_a, rows_b, sem, sem_s) = rest

        c = lax.axis_index("c")
        s = lax.axis_index("s")
        tid = c * NS + s

        pltpu.sync_copy(src_hbm.at[tid], srct_v)
        pltpu.sync_copy(dst_hbm.at[tid], dstt_v)

        pltpu.sync_copy(
            zeros_hbm.at[pl.ds(s * ROWS_PER_TILE, ROWS_PER_TILE)],
            acc_sh.at[pl.ds(s * ROWS_PER_TILE, ROWS_PER_TILE)],
        )
        if compute_deg:
            for i in range(CHUNK // 16):
                ones_v[pl.ds(i * 16, 16)] = jnp.ones((16,), jnp.float32)

            @pl.when(s == 0)
            def _():
                pltpu.sync_copy(zdeg_hbm.at[0], deg_sh)

        plsc.subcore_barrier()

        def gather(i, buf):
            pltpu.async_copy(x_hbm.at[srct_v.at[pl.ds(i * CHUNK, CHUNK)]], buf, sem)

        def wait_rows(buf):
            pltpu.make_async_copy(zeros_hbm.at[pl.ds(0, CHUNK)], buf, sem).wait()

        def scatter(i, buf):
            pltpu.async_copy(buf, acc_sh.at[dstt_v.at[i]], sem_s, add=True)
            if compute_deg:
                pltpu.sync_copy(ones_v, deg_sh.at[dstt_v.at[i]], add=True)

        def wait_scatter(buf):
            pltpu.make_async_copy(buf, acc_sh.at[dstt_v.at[0]], sem_s).wait()

        gather(0, rows_a)

        def chunk_body(i, _):
            a = 2 * i
            b = a + 1
            nxt = a + 2
            wait_rows(rows_a)
            gather(b, rows_b)
            scatter(a, rows_a)
            wait_rows(rows_b)
            wait_scatter(rows_a)

            @pl.when(nxt < N_CHUNKS)
            def _():
                gather(nxt, rows_a)

            scatter(b, rows_b)
            wait_scatter(rows_b)
            return ()

        lax.fori_loop(0, N_CHUNKS // 2, chunk_body, ())
        if N_CHUNKS % 2 == 1:
            wait_rows(rows_a)
            scatter(N_CHUNKS - 1, rows_a)
            wait_scatter(rows_a)

        plsc.subcore_barrier()

        row0 = s * ROWS_PER_TILE
        pltpu.sync_copy(
            acc_sh.at[pl.ds(row0, ROWS_PER_TILE)],
            part_hbm.at[pl.ds(c * N_PAD + row0, ROWS_PER_TILE)],
        )
        if compute_deg:
            @pl.when(s == 0)
            def _():
                pltpu.sync_copy(deg_sh, deg_hbm.at[c])

    return pl.kernel(
        body,
        out_type=tuple(out_type) if compute_deg else out_type[0],
        mesh=mesh,
        scratch_types=scratch,
    )


_sc_agg_with_deg = _make_sc_agg(True)
_sc_agg = _make_sc_agg(False)


ROW_BLK = ROWS_PER_TILE
GRID = N_PAD // ROW_BLK


def _tc_xr_body(x, wr, b, out):
    out[...] = (
        jnp.dot(x[...], wr[...], preferred_element_type=jnp.float32) + b[...]
    )


_tc_xr = pl.pallas_call(
    _tc_xr_body,
    out_shape=jax.ShapeDtypeStruct((N_PAD, D), jnp.float32),
    grid=(GRID,),
    in_specs=[
        pl.BlockSpec((ROW_BLK, D), lambda i: (i, 0)),
        pl.BlockSpec((D, D), lambda i: (0, 0)),
        pl.BlockSpec((1, D), lambda i: (0, 0)),
    ],
    out_specs=pl.BlockSpec((ROW_BLK, D), lambda i: (i, 0)),
)


def _make_tc_combine(relu: bool):
    def tc_body(pa, pb, da, db, xr, wl, out):
        deg = jnp.maximum(da[...] + db[...], 1.0)
        mean = (pa[...] + pb[...]) / deg
        y = jnp.dot(mean, wl[...], preferred_element_type=jnp.float32) + xr[...]
        if relu:
            y = jnp.maximum(y, 0.0)
        out[...] = y

    return pl.pallas_call(
        tc_body,
        out_shape=jax.ShapeDtypeStruct((N_PAD, D), jnp.float32),
        grid=(GRID,),
        in_specs=[
            pl.BlockSpec((ROW_BLK, D), lambda i: (i, 0)),
            pl.BlockSpec((ROW_BLK, D), lambda i: (i + GRID, 0)),
            pl.BlockSpec((ROW_BLK, 1), lambda i: (i, 0)),
            pl.BlockSpec((ROW_BLK, 1), lambda i: (i + GRID, 0)),
            pl.BlockSpec((ROW_BLK, D), lambda i: (i, 0)),
            pl.BlockSpec((D, D), lambda i: (0, 0)),
        ],
        out_specs=pl.BlockSpec((ROW_BLK, D), lambda i: (i, 0)),
    )


_tc_relu = _make_tc_combine(True)
_tc_lin = _make_tc_combine(False)


@jax.jit
def _impl(x, edge_index, W_l1, b1, W_r1, W_l2, b2, W_r2):
    src = edge_index[0].reshape(NC * NS, EDGES_PER_TILE)
    dst = edge_index[1].reshape(NC * NS, N_CHUNKS, CHUNK)
    x_pad = jnp.concatenate([x, jnp.zeros((N_PAD - N_NODES, D), jnp.float32)])
    zeros2d = jnp.zeros((N_PAD, D), jnp.float32)
    zdeg = jnp.zeros((1, N_PAD), jnp.float32)
    b1r = b1.reshape(1, D)
    b2r = b2.reshape(1, D)

    xr1 = _tc_xr(x_pad, W_r1, b1r)
    part1, deg = _sc_agg_with_deg(x_pad, src, dst, zeros2d, zdeg)
    deg2 = deg.reshape(NC * N_PAD, 1)
    h = _tc_relu(part1, part1, deg2, deg2, xr1, W_l1)
    hr2 = _tc_xr(h, W_r2, b2r)
    part2 = _sc_agg(h, src, dst, zeros2d, zdeg)
    out = _tc_lin(part2, part2, deg2, deg2, hr2, W_l2)
    return lax.slice(out, (0, 0), (N_NODES, D))


def kernel(x, edge_index, W_l1, b1, W_r1, W_l2, b2, W_r2):
    return _impl(x, edge_index, W_l1, b1, W_r1, W_l2, b2, W_r2)


# speedup vs baseline: 1.0044x; 1.0044x over previous
import functools

import jax
import jax.numpy as jnp
from jax import lax
from jax.experimental import pallas as pl
from jax.experimental.pallas import tpu as pltpu
from jax.experimental.pallas import tpu_sc as plsc

N_NODES = 10000
N_EDGES = 320000
D = 128

NC = 2
NS = 16
EDGES_PER_CORE = N_EDGES // NC
EDGES_PER_TILE = EDGES_PER_CORE // NS
CHUNK = 80
N_CHUNKS = EDGES_PER_TILE // CHUNK
ROWS_PER_TILE = 632
N_PAD = ROWS_PER_TILE * NS


def _make_sc_agg(compute_deg: bool):
    mesh = plsc.VectorSubcoreMesh(core_axis_name="c", subcore_axis_name="s")

    out_type = [jax.ShapeDtypeStruct((NC * N_PAD, D), jnp.float32)]
    if compute_deg:
        out_type.append(jax.ShapeDtypeStruct((NC, N_PAD), jnp.float32))

    scratch = [
        pltpu.VMEM_SHARED((N_PAD, D), jnp.float32),
        pltpu.VMEM((EDGES_PER_TILE,), jnp.int32),
        pltpu.VMEM((N_CHUNKS, CHUNK), jnp.int32),
        pltpu.VMEM((CHUNK, D), jnp.float32),
        pltpu.VMEM((CHUNK, D), jnp.float32),
        pltpu.SemaphoreType.DMA,
        pltpu.SemaphoreType.DMA,
    ]
    if compute_deg:
        scratch += [
            pltpu.VMEM_SHARED((N_PAD,), jnp.float32),
            pltpu.VMEM((CHUNK,), jnp.float32),
        ]

    def body(x_hbm, src_hbm, dst_hbm, zeros_hbm, zdeg_hbm, part_hbm, *rest):
        if compute_deg:
            deg_hbm = rest[0]
            acc_sh, srct_v, dstt_v, rows_a, rows_b, sem, sem_s, deg_sh, ones_v = rest[1:]
        else:
            (acc_sh, srct_v, dstt_v, rows_a, rows_b, sem, sem_s) = rest

        c = lax.axis_index("c")
        s = lax.axis_index("s")
        tid = c * NS + s

        pltpu.sync_copy(src_hbm.at[tid], srct_v)
        pltpu.sync_copy(dst_hbm.at[tid], dstt_v)

        pltpu.async_copy(
            x_hbm.at[srct_v.at[pl.ds(0, CHUNK)]], rows_a, sem)

        pltpu.sync_copy(
            zeros_hbm.at[pl.ds(s * ROWS_PER_TILE, ROWS_PER_TILE)],
            acc_sh.at[pl.ds(s * ROWS_PER_TILE, ROWS_PER_TILE)],
        )
        if compute_deg:
            for i in range(CHUNK // 16):
                ones_v[pl.ds(i * 16, 16)] = jnp.ones((16,), jnp.float32)

            @pl.when(s == 0)
            def _():
                pltpu.sync_copy(zdeg_hbm.at[0], deg_sh)

        plsc.subcore_barrier()

        def gather(i, buf):
            pltpu.async_copy(x_hbm.at[srct_v.at[pl.ds(i * CHUNK, CHUNK)]], buf, sem)

        def wait_rows(buf):
            pltpu.make_async_copy(zeros_hbm.at[pl.ds(0, CHUNK)], buf, sem).wait()

        def scatter(i, buf):
            pltpu.async_copy(buf, acc_sh.at[dstt_v.at[i]], sem_s, add=True)
            if compute_deg:
                pltpu.sync_copy(ones_v, deg_sh.at[dstt_v.at[i]], add=True)

        def wait_scatter(buf):
            pltpu.make_async_copy(buf, acc_sh.at[dstt_v.at[0]], sem_s).wait()

        def chunk_body(i, _):
            a = 2 * i
            b = a + 1
            nxt = a + 2
            wait_rows(rows_a)
            gather(b, rows_b)
            scatter(a, rows_a)
            wait_rows(rows_b)
            wait_scatter(rows_a)

            @pl.when(nxt < N_CHUNKS)
            def _():
                gather(nxt, rows_a)

            scatter(b, rows_b)
            wait_scatter(rows_b)
            return ()

        lax.fori_loop(0, N_CHUNKS // 2, chunk_body, ())
        if N_CHUNKS % 2 == 1:
            wait_rows(rows_a)
            scatter(N_CHUNKS - 1, rows_a)
            wait_scatter(rows_a)

        plsc.subcore_barrier()

        row0 = s * ROWS_PER_TILE
        pltpu.sync_copy(
            acc_sh.at[pl.ds(row0, ROWS_PER_TILE)],
            part_hbm.at[pl.ds(c * N_PAD + row0, ROWS_PER_TILE)],
        )
        if compute_deg:
            @pl.when(s == 0)
            def _():
                pltpu.sync_copy(deg_sh, deg_hbm.at[c])

    return pl.kernel(
        body,
        out_type=tuple(out_type) if compute_deg else out_type[0],
        mesh=mesh,
        scratch_types=scratch,
    )


_sc_agg_with_deg = _make_sc_agg(True)
_sc_agg = _make_sc_agg(False)


ROW_BLK = ROWS_PER_TILE
GRID = N_PAD // ROW_BLK


def _tc_xr_body(x, wr, b, out):
    out[...] = (
        jnp.dot(x[...], wr[...], preferred_element_type=jnp.float32) + b[...]
    )


_tc_xr = pl.pallas_call(
    _tc_xr_body,
    out_shape=jax.ShapeDtypeStruct((N_PAD, D), jnp.float32),
    grid=(GRID,),
    in_specs=[
        pl.BlockSpec((ROW_BLK, D), lambda i: (i, 0)),
        pl.BlockSpec((D, D), lambda i: (0, 0)),
        pl.BlockSpec((1, D), lambda i: (0, 0)),
    ],
    out_specs=pl.BlockSpec((ROW_BLK, D), lambda i: (i, 0)),
)


def _make_tc_combine(relu: bool):
    def tc_body(pa, pb, da, db, xr, wl, out):
        deg = jnp.maximum(da[...] + db[...], 1.0)
        mean = (pa[...] + pb[...]) / deg
        y = jnp.dot(mean, wl[...], preferred_element_type=jnp.float32) + xr[...]
        if relu:
            y = jnp.maximum(y, 0.0)
        out[...] = y

    return pl.pallas_call(
        tc_body,
        out_shape=jax.ShapeDtypeStruct((N_PAD, D), jnp.float32),
        grid=(GRID,),
        in_specs=[
            pl.BlockSpec((ROW_BLK, D), lambda i: (i, 0)),
            pl.BlockSpec((ROW_BLK, D), lambda i: (i + GRID, 0)),
            pl.BlockSpec((ROW_BLK, 1), lambda i: (i, 0)),
            pl.BlockSpec((ROW_BLK, 1), lambda i: (i + GRID, 0)),
            pl.BlockSpec((ROW_BLK, D), lambda i: (i, 0)),
            pl.BlockSpec((D, D), lambda i: (0, 0)),
        ],
        out_specs=pl.BlockSpec((ROW_BLK, D), lambda i: (i, 0)),
    )


_tc_relu = _make_tc_combine(True)
_tc_lin = _make_tc_combine(False)


@jax.jit
def _impl(x, edge_index, W_l1, b1, W_r1, W_l2, b2, W_r2):
    src = edge_index[0].reshape(NC * NS, EDGES_PER_TILE)
    dst = edge_index[1].reshape(NC * NS, N_CHUNKS, CHUNK)
    x_pad = jnp.concatenate([x, jnp.zeros((N_PAD - N_NODES, D), jnp.float32)])
    zeros2d = jnp.zeros((N_PAD, D), jnp.float32)
    zdeg = jnp.zeros((1, N_PAD), jnp.float32)
    b1r = b1.reshape(1, D)
    b2r = b2.reshape(1, D)

    xr1 = _tc_xr(x_pad, W_r1, b1r)
    part1, deg = _sc_agg_with_deg(x_pad, src, dst, zeros2d, zdeg)
    deg2 = deg.reshape(NC * N_PAD, 1)
    h = _tc_relu(part1, part1, deg2, deg2, xr1, W_l1)
    hr2 = _tc_xr(h, W_r2, b2r)
    part2 = _sc_agg(h, src, dst, zeros2d, zdeg)
    out = _tc_lin(part2, part2, deg2, deg2, hr2, W_l2)
    return lax.slice(out, (0, 0), (N_NODES, D))


def kernel(x, edge_index, W_l1, b1, W_r1, W_l2, b2, W_r2):
    return _impl(x, edge_index, W_l1, b1, W_r1, W_l2, b2, W_r2)
